# baseline (speedup 1.0000x reference)
import functools
import math

import jax
import jax.numpy as jnp
from jax import lax
from jax.experimental import pallas as pl
from jax.experimental.pallas import tpu as pltpu
from jax.experimental.pallas import tpu_sc as plsc

SIZE = 32000
PADDING_IDX = 0
SMOOTHING = 0.1
CONFIDENCE = 1.0 - SMOOTHING
SMOOTH_VAL = SMOOTHING / (SIZE - 2)
K_ROW = CONFIDENCE * math.log(CONFIDENCE) + SMOOTH_VAL * (SIZE - 2) * math.log(SMOOTH_VAL)

N = 4096
ROW_BLOCK = 128

TC_ROWS = 2048
SC_ROWS = N - TC_ROWS
NUM_BLOCKS = TC_ROWS // ROW_BLOCK

SC_CORES = 2
SC_SUBCORES = 16
SC_WORKERS = SC_CORES * SC_SUBCORES
SC_LANES = 16
DENSE_PER_W = SC_ROWS // SC_WORKERS


def _row_accumulate(buf):
    def inner(k, accs):
        accs = list(accs)
        base = k * 640
        for u in range(40):
            accs[u % 8] = accs[u % 8] + buf[pl.ds(base + u * 16, 16)]
        return tuple(accs)

    z = jnp.zeros((SC_LANES,), jnp.float32)
    accs = lax.fori_loop(0, SIZE // 640, inner, (z,) * 8)
    return (((accs[0] + accs[1]) + (accs[2] + accs[3]))
            + ((accs[4] + accs[5]) + (accs[6] + accs[7])))


def _sc_body(x_hbm, tgt_hbm, out_hbm, tgt_d, buf0, buf1, buf2, acc_v,
             sem0, sem1, sem2):
    wid = lax.axis_index("s") * SC_CORES + lax.axis_index("c")
    lane = lax.broadcasted_iota(jnp.int32, (SC_LANES,), 0)
    dstart = TC_ROWS + wid * DENSE_PER_W
    pltpu.sync_copy(tgt_hbm.at[pl.ds(dstart, DENSE_PER_W)], tgt_d)

    krow = jnp.full((SC_LANES,), jnp.float32(K_ROW / SC_LANES))
    neg_s = jnp.float32(-SMOOTH_VAL)
    cg16 = jnp.float32(-(CONFIDENCE - SMOOTH_VAL) / SC_LANES)

    def row_contrib(buf, j):
        row_acc = _row_accumulate(buf)
        c0 = jnp.where(lane == 0, buf[pl.ds(0, SC_LANES)], 0.0)
        jv = jnp.full((SC_LANES,), 0, jnp.int32) + j
        t_b = plsc.load_gather(tgt_d, [jv])
        gval = plsc.load_gather(buf, [t_b])
        contrib = krow + neg_s * (row_acc - c0) + cg16 * gval
        return jnp.where(t_b != PADDING_IDX, contrib, 0.0)

    def start_dma(j, buf, sem):
        pltpu.async_copy(x_hbm.at[dstart + j], buf, sem)

    def wait_dma(buf, sem):
        pltpu.make_async_copy(x_hbm.at[0], buf, sem).wait()

    start_dma(0, buf0, sem0)
    start_dma(1, buf1, sem1)

    def outer(j3, carry):
        (dacc,) = carry
        r = 3 * j3

        @pl.when(r + 2 < DENSE_PER_W)
        def _():
            start_dma(r + 2, buf2, sem2)

        wait_dma(buf0, sem0)
        dacc = dacc + row_contrib(buf0, r)

        @pl.when(r + 3 < DENSE_PER_W)
        def _():
            start_dma(r + 3, buf0, sem0)

        wait_dma(buf1, sem1)
        dacc = dacc + row_contrib(buf1, r + 1)

        @pl.when(r + 4 < DENSE_PER_W)
        def _():
            start_dma(r + 4, buf1, sem1)

        wait_dma(buf2, sem2)
        dacc = dacc + row_contrib(buf2, r + 2)
        return (dacc,)

    z = jnp.zeros((SC_LANES,), jnp.float32)
    (dacc,) = lax.fori_loop(0, DENSE_PER_W // 3, outer, (z,))

    rem_base = (DENSE_PER_W // 3) * 3
    rem_bufs = (buf0, buf1, buf2)
    rem_sems = (sem0, sem1, sem2)
    for e in range(DENSE_PER_W % 3):
        r = rem_base + e
        wait_dma(rem_bufs[r % 3], rem_sems[r % 3])
        dacc = dacc + row_contrib(rem_bufs[r % 3], r)

    acc_v[...] = dacc
    pltpu.sync_copy(acc_v, out_hbm.at[wid])


@functools.cache
def _sc_kernel():
    return pl.kernel(
        _sc_body,
        mesh=plsc.VectorSubcoreMesh(core_axis_name="c", subcore_axis_name="s"),
        compiler_params=pltpu.CompilerParams(needs_layout_passes=False),
        out_type=jax.ShapeDtypeStruct((SC_WORKERS, SC_LANES), jnp.float32),
        scratch_types=[
            pltpu.VMEM((DENSE_PER_W,), jnp.int32),
            pltpu.VMEM((SIZE,), jnp.float32),
            pltpu.VMEM((SIZE,), jnp.float32),
            pltpu.VMEM((SIZE,), jnp.float32),
            pltpu.VMEM((SC_LANES,), jnp.float32),
            pltpu.SemaphoreType.DMA,
            pltpu.SemaphoreType.DMA,
            pltpu.SemaphoreType.DMA,
        ],
    )


def _tc_body(x_ref, t_ref, out_ref):
    i = pl.program_id(0)
    t = t_ref[...]
    m = (t != PADDING_IDX).astype(jnp.float32)
    xb = x_ref[...]
    rowsums = jnp.sum(xb, axis=1, keepdims=True)
    cols = lax.broadcasted_iota(jnp.int32, (ROW_BLOCK, SIZE), 1)
    gvals = jnp.sum(jnp.where(cols == t, xb, 0.0), axis=1, keepdims=True)
    contrib = (
        jnp.float32(K_ROW) * jnp.sum(m)
        - jnp.float32(SMOOTH_VAL) * jnp.sum(m * rowsums)
        + jnp.float32(SMOOTH_VAL) * jnp.sum(m * xb[:, 0:1])
        - jnp.float32(CONFIDENCE - SMOOTH_VAL) * jnp.sum(m * gvals)
    )

    @pl.when(i == 0)
    def _():
        out_ref[0, 0] = 0.0

    out_ref[0, 0] += contrib


def _tc_reduce(x, t2d):
    return pl.pallas_call(
        _tc_body,
        grid=(NUM_BLOCKS,),
        in_specs=[
            pl.BlockSpec((ROW_BLOCK, SIZE), lambda i: (i, 0)),
            pl.BlockSpec((ROW_BLOCK, 1), lambda i: (i, 0)),
        ],
        out_specs=pl.BlockSpec((1, 1), lambda i: (0, 0), memory_space=pltpu.SMEM),
        out_shape=jax.ShapeDtypeStruct((1, 1), jnp.float32),
        compiler_params=pltpu.CompilerParams(
            dimension_semantics=("arbitrary",),
        ),
    )(x, t2d)


def kernel(x, target):
    target = target.astype(jnp.int32)
    sc_parts = _sc_kernel()(x, target)
    tc_out = _tc_reduce(x, target.reshape(N, 1))
    return tc_out[0, 0] + jnp.sum(sc_parts)


# speedup vs baseline: 1.0017x; 1.0017x over previous
import functools
import math

import jax
import jax.numpy as jnp
from jax import lax
from jax.experimental import pallas as pl
from jax.experimental.pallas import tpu as pltpu
from jax.experimental.pallas import tpu_sc as plsc

SIZE = 32000
PADDING_IDX = 0
SMOOTHING = 0.1
CONFIDENCE = 1.0 - SMOOTHING
SMOOTH_VAL = SMOOTHING / (SIZE - 2)
K_ROW = CONFIDENCE * math.log(CONFIDENCE) + SMOOTH_VAL * (SIZE - 2) * math.log(SMOOTH_VAL)

N = 4096
ROW_BLOCK = 128

TC_ROWS = 1536
SC_ROWS = N - TC_ROWS
NUM_BLOCKS = TC_ROWS // ROW_BLOCK

SC_CORES = 2
SC_SUBCORES = 16
SC_WORKERS = SC_CORES * SC_SUBCORES
SC_LANES = 16
DENSE_PER_W = SC_ROWS // SC_WORKERS


def _row_accumulate(buf):
    def inner(k, accs):
        accs = list(accs)
        base = k * 640
        for u in range(40):
            accs[u % 8] = accs[u % 8] + buf[pl.ds(base + u * 16, 16)]
        return tuple(accs)

    z = jnp.zeros((SC_LANES,), jnp.float32)
    accs = lax.fori_loop(0, SIZE // 640, inner, (z,) * 8)
    return (((accs[0] + accs[1]) + (accs[2] + accs[3]))
            + ((accs[4] + accs[5]) + (accs[6] + accs[7])))


def _sc_body(x_hbm, tgt_hbm, out_hbm, tgt_d, buf0, buf1, buf2, acc_v,
             sem0, sem1, sem2):
    wid = lax.axis_index("s") * SC_CORES + lax.axis_index("c")
    lane = lax.broadcasted_iota(jnp.int32, (SC_LANES,), 0)
    dstart = TC_ROWS + wid * DENSE_PER_W
    pltpu.sync_copy(tgt_hbm.at[pl.ds(dstart, DENSE_PER_W)], tgt_d)

    krow = jnp.full((SC_LANES,), jnp.float32(K_ROW / SC_LANES))
    neg_s = jnp.float32(-SMOOTH_VAL)
    cg16 = jnp.float32(-(CONFIDENCE - SMOOTH_VAL) / SC_LANES)

    def row_contrib(buf, j):
        row_acc = _row_accumulate(buf)
        c0 = jnp.where(lane == 0, buf[pl.ds(0, SC_LANES)], 0.0)
        jv = jnp.full((SC_LANES,), 0, jnp.int32) + j
        t_b = plsc.load_gather(tgt_d, [jv])
        gval = plsc.load_gather(buf, [t_b])
        contrib = krow + neg_s * (row_acc - c0) + cg16 * gval
        return jnp.where(t_b != PADDING_IDX, contrib, 0.0)

    def start_dma(j, buf, sem):
        pltpu.async_copy(x_hbm.at[dstart + j], buf, sem)

    def wait_dma(buf, sem):
        pltpu.make_async_copy(x_hbm.at[0], buf, sem).wait()

    start_dma(0, buf0, sem0)
    start_dma(1, buf1, sem1)

    def outer(j3, carry):
        (dacc,) = carry
        r = 3 * j3

        @pl.when(r + 2 < DENSE_PER_W)
        def _():
            start_dma(r + 2, buf2, sem2)

        wait_dma(buf0, sem0)
        dacc = dacc + row_contrib(buf0, r)

        @pl.when(r + 3 < DENSE_PER_W)
        def _():
            start_dma(r + 3, buf0, sem0)

        wait_dma(buf1, sem1)
        dacc = dacc + row_contrib(buf1, r + 1)

        @pl.when(r + 4 < DENSE_PER_W)
        def _():
            start_dma(r + 4, buf1, sem1)

        wait_dma(buf2, sem2)
        dacc = dacc + row_contrib(buf2, r + 2)
        return (dacc,)

    z = jnp.zeros((SC_LANES,), jnp.float32)
    (dacc,) = lax.fori_loop(0, DENSE_PER_W // 3, outer, (z,))

    rem_base = (DENSE_PER_W // 3) * 3
    rem_bufs = (buf0, buf1, buf2)
    rem_sems = (sem0, sem1, sem2)
    for e in range(DENSE_PER_W % 3):
        r = rem_base + e
        wait_dma(rem_bufs[r % 3], rem_sems[r % 3])
        dacc = dacc + row_contrib(rem_bufs[r % 3], r)

    acc_v[...] = dacc
    pltpu.sync_copy(acc_v, out_hbm.at[wid])


@functools.cache
def _sc_kernel():
    return pl.kernel(
        _sc_body,
        mesh=plsc.VectorSubcoreMesh(core_axis_name="c", subcore_axis_name="s"),
        compiler_params=pltpu.CompilerParams(needs_layout_passes=False),
        out_type=jax.ShapeDtypeStruct((SC_WORKERS, SC_LANES), jnp.float32),
        scratch_types=[
            pltpu.VMEM((DENSE_PER_W,), jnp.int32),
            pltpu.VMEM((SIZE,), jnp.float32),
            pltpu.VMEM((SIZE,), jnp.float32),
            pltpu.VMEM((SIZE,), jnp.float32),
            pltpu.VMEM((SC_LANES,), jnp.float32),
            pltpu.SemaphoreType.DMA,
            pltpu.SemaphoreType.DMA,
            pltpu.SemaphoreType.DMA,
        ],
    )


def _tc_body(x_ref, t_ref, out_ref):
    i = pl.program_id(0)
    t = t_ref[...]
    m = (t != PADDING_IDX).astype(jnp.float32)
    xb = x_ref[...]
    rowsums = jnp.sum(xb, axis=1, keepdims=True)
    cols = lax.broadcasted_iota(jnp.int32, (ROW_BLOCK, SIZE), 1)
    gvals = jnp.sum(jnp.where(cols == t, xb, 0.0), axis=1, keepdims=True)
    contrib = (
        jnp.float32(K_ROW) * jnp.sum(m)
        - jnp.float32(SMOOTH_VAL) * jnp.sum(m * rowsums)
        + jnp.float32(SMOOTH_VAL) * jnp.sum(m * xb[:, 0:1])
        - jnp.float32(CONFIDENCE - SMOOTH_VAL) * jnp.sum(m * gvals)
    )

    @pl.when(i == 0)
    def _():
        out_ref[0, 0] = 0.0

    out_ref[0, 0] += contrib


def _tc_reduce(x, t2d):
    return pl.pallas_call(
        _tc_body,
        grid=(NUM_BLOCKS,),
        in_specs=[
            pl.BlockSpec((ROW_BLOCK, SIZE), lambda i: (i, 0)),
            pl.BlockSpec((ROW_BLOCK, 1), lambda i: (i, 0)),
        ],
        out_specs=pl.BlockSpec((1, 1), lambda i: (0, 0), memory_space=pltpu.SMEM),
        out_shape=jax.ShapeDtypeStruct((1, 1), jnp.float32),
        compiler_params=pltpu.CompilerParams(
            dimension_semantics=("arbitrary",),
        ),
    )(x, t2d)


def kernel(x, target):
    target = target.astype(jnp.int32)
    sc_parts = _sc_kernel()(x, target)
    tc_out = _tc_reduce(x, target.reshape(N, 1))
    return tc_out[0, 0] + jnp.sum(sc_parts)


# speedup vs baseline: 1.0039x; 1.0021x over previous
import functools
import math

import jax
import jax.numpy as jnp
from jax import lax
from jax.experimental import pallas as pl
from jax.experimental.pallas import tpu as pltpu
from jax.experimental.pallas import tpu_sc as plsc

SIZE = 32000
PADDING_IDX = 0
SMOOTHING = 0.1
CONFIDENCE = 1.0 - SMOOTHING
SMOOTH_VAL = SMOOTHING / (SIZE - 2)
K_ROW = CONFIDENCE * math.log(CONFIDENCE) + SMOOTH_VAL * (SIZE - 2) * math.log(SMOOTH_VAL)

N = 4096
ROW_BLOCK = 128

TC_ROWS = 1792
SC_ROWS = N - TC_ROWS
NUM_BLOCKS = TC_ROWS // ROW_BLOCK

SC_CORES = 2
SC_SUBCORES = 16
SC_WORKERS = SC_CORES * SC_SUBCORES
SC_LANES = 16
DENSE_PER_W = SC_ROWS // SC_WORKERS


def _row_accumulate(buf):
    def inner(k, accs):
        accs = list(accs)
        base = k * 640
        for u in range(40):
            accs[u % 8] = accs[u % 8] + buf[pl.ds(base + u * 16, 16)]
        return tuple(accs)

    z = jnp.zeros((SC_LANES,), jnp.float32)
    accs = lax.fori_loop(0, SIZE // 640, inner, (z,) * 8)
    return (((accs[0] + accs[1]) + (accs[2] + accs[3]))
            + ((accs[4] + accs[5]) + (accs[6] + accs[7])))


def _sc_body(x_hbm, tgt_hbm, out_hbm, tgt_d, buf0, buf1, buf2, acc_v,
             sem0, sem1, sem2):
    wid = lax.axis_index("s") * SC_CORES + lax.axis_index("c")
    lane = lax.broadcasted_iota(jnp.int32, (SC_LANES,), 0)
    dstart = TC_ROWS + wid * DENSE_PER_W
    pltpu.sync_copy(tgt_hbm.at[pl.ds(dstart, DENSE_PER_W)], tgt_d)

    krow = jnp.full((SC_LANES,), jnp.float32(K_ROW / SC_LANES))
    neg_s = jnp.float32(-SMOOTH_VAL)
    cg16 = jnp.float32(-(CONFIDENCE - SMOOTH_VAL) / SC_LANES)

    def row_contrib(buf, j):
        row_acc = _row_accumulate(buf)
        c0 = jnp.where(lane == 0, buf[pl.ds(0, SC_LANES)], 0.0)
        jv = jnp.full((SC_LANES,), 0, jnp.int32) + j
        t_b = plsc.load_gather(tgt_d, [jv])
        gval = plsc.load_gather(buf, [t_b])
        contrib = krow + neg_s * (row_acc - c0) + cg16 * gval
        return jnp.where(t_b != PADDING_IDX, contrib, 0.0)

    def start_dma(j, buf, sem):
        pltpu.async_copy(x_hbm.at[dstart + j], buf, sem)

    def wait_dma(buf, sem):
        pltpu.make_async_copy(x_hbm.at[0], buf, sem).wait()

    start_dma(0, buf0, sem0)
    start_dma(1, buf1, sem1)

    def outer(j3, carry):
        (dacc,) = carry
        r = 3 * j3

        @pl.when(r + 2 < DENSE_PER_W)
        def _():
            start_dma(r + 2, buf2, sem2)

        wait_dma(buf0, sem0)
        dacc = dacc + row_contrib(buf0, r)

        @pl.when(r + 3 < DENSE_PER_W)
        def _():
            start_dma(r + 3, buf0, sem0)

        wait_dma(buf1, sem1)
        dacc = dacc + row_contrib(buf1, r + 1)

        @pl.when(r + 4 < DENSE_PER_W)
        def _():
            start_dma(r + 4, buf1, sem1)

        wait_dma(buf2, sem2)
        dacc = dacc + row_contrib(buf2, r + 2)
        return (dacc,)

    z = jnp.zeros((SC_LANES,), jnp.float32)
    (dacc,) = lax.fori_loop(0, DENSE_PER_W // 3, outer, (z,))

    rem_base = (DENSE_PER_W // 3) * 3
    rem_bufs = (buf0, buf1, buf2)
    rem_sems = (sem0, sem1, sem2)
    for e in range(DENSE_PER_W % 3):
        r = rem_base + e
        wait_dma(rem_bufs[r % 3], rem_sems[r % 3])
        dacc = dacc + row_contrib(rem_bufs[r % 3], r)

    acc_v[...] = dacc
    pltpu.sync_copy(acc_v, out_hbm.at[wid])


@functools.cache
def _sc_kernel():
    return pl.kernel(
        _sc_body,
        mesh=plsc.VectorSubcoreMesh(core_axis_name="c", subcore_axis_name="s"),
        compiler_params=pltpu.CompilerParams(needs_layout_passes=False),
        out_type=jax.ShapeDtypeStruct((SC_WORKERS, SC_LANES), jnp.float32),
        scratch_types=[
            pltpu.VMEM((DENSE_PER_W,), jnp.int32),
            pltpu.VMEM((SIZE,), jnp.float32),
            pltpu.VMEM((SIZE,), jnp.float32),
            pltpu.VMEM((SIZE,), jnp.float32),
            pltpu.VMEM((SC_LANES,), jnp.float32),
            pltpu.SemaphoreType.DMA,
            pltpu.SemaphoreType.DMA,
            pltpu.SemaphoreType.DMA,
        ],
    )


def _tc_body(x_ref, t_ref, out_ref):
    i = pl.program_id(0)
    t = t_ref[...]
    m = (t != PADDING_IDX).astype(jnp.float32)
    xb = x_ref[...]
    rowsums = jnp.sum(xb, axis=1, keepdims=True)
    cols = lax.broadcasted_iota(jnp.int32, (ROW_BLOCK, SIZE), 1)
    gvals = jnp.sum(jnp.where(cols == t, xb, 0.0), axis=1, keepdims=True)
    contrib = (
        jnp.float32(K_ROW) * jnp.sum(m)
        - jnp.float32(SMOOTH_VAL) * jnp.sum(m * rowsums)
        + jnp.float32(SMOOTH_VAL) * jnp.sum(m * xb[:, 0:1])
        - jnp.float32(CONFIDENCE - SMOOTH_VAL) * jnp.sum(m * gvals)
    )

    @pl.when(i == 0)
    def _():
        out_ref[0, 0] = 0.0

    out_ref[0, 0] += contrib


def _tc_reduce(x, t2d):
    return pl.pallas_call(
        _tc_body,
        grid=(NUM_BLOCKS,),
        in_specs=[
            pl.BlockSpec((ROW_BLOCK, SIZE), lambda i: (i, 0)),
            pl.BlockSpec((ROW_BLOCK, 1), lambda i: (i, 0)),
        ],
        out_specs=pl.BlockSpec((1, 1), lambda i: (0, 0), memory_space=pltpu.SMEM),
        out_shape=jax.ShapeDtypeStruct((1, 1), jnp.float32),
        compiler_params=pltpu.CompilerParams(
            dimension_semantics=("arbitrary",),
        ),
    )(x, t2d)


def kernel(x, target):
    target = target.astype(jnp.int32)
    sc_parts = _sc_kernel()(x, target)
    tc_out = _tc_reduce(x, target.reshape(N, 1))
    return tc_out[0, 0] + jnp.sum(sc_parts)
